# baseline (speedup 1.0000x reference)
import functools

import jax
import jax.numpy as jnp
from jax import lax
from jax.experimental import pallas as pl
from jax.experimental.pallas import tpu as pltpu
from jax.experimental.pallas import tpu_sc as plsc

N_NODES = 100000
N_EDGES = 1600000
D_H = 128
D_E = 16
N_SEG = 5000
SEG_PAD = 5120
NC, NS = 2, 16
CN = 80
NCHN = N_NODES // CN
CE = 128
NCHE = N_EDGES // CE

ROW_N = 272
ROW_E = 40

_mesh = plsc.VectorSubcoreMesh(core_axis_name="c", subcore_axis_name="s")

_GATHER_DNUMS = lax.GatherDimensionNumbers(
    offset_dims=(), collapsed_slice_dims=(0,), start_index_map=(0,))


def _shuffle(x, idx):
    return lax.gather(x, idx[:, None], _GATHER_DNUMS, (1,),
                      mode=lax.GatherScatterMode.PROMISE_IN_BOUNDS)


@functools.partial(
    pl.kernel,
    out_type=jax.ShapeDtypeStruct((NC, SEG_PAD, ROW_N), jnp.float32),
    mesh=_mesh,
    compiler_params=pltpu.CompilerParams(use_tc_tiling_on_sc=False),
    scratch_types=[
        pltpu.VMEM((CN,), jnp.int32),
        pltpu.VMEM((CN, D_H), jnp.float32),
        pltpu.VMEM((CN, 16), jnp.float32),
        pltpu.VMEM((CN, ROW_N), jnp.float32),
        pltpu.VMEM_SHARED((SEG_PAD, ROW_N), jnp.float32),
        pltpu.SemaphoreType.DMA,
    ],
)
def _node_stats(h_hbm, zp_hbm, bid_hbm, zn_hbm, nacc_out,
                segs, hbuf, zbuf, nrows, acc_n, sem):
    c = lax.axis_index("c")
    s = lax.axis_index("s")
    rpt = SEG_PAD // NS
    pltpu.sync_copy(zn_hbm.at[pl.ds(s * rpt, rpt), :],
                    acc_n.at[pl.ds(s * rpt, rpt), :])
    iota = lax.iota(jnp.int32, 16)
    cvec_n = jnp.where(iota == 6, 1.0, 0.0).astype(jnp.float32)
    sh3 = (iota - 3) & 15

    plsc.subcore_barrier()

    npc = NCHN // NC

    @pl.loop(c * npc + s, (c + 1) * npc, step=NS)
    def _(ch):
        base = ch * CN
        pltpu.sync_copy(bid_hbm.at[pl.ds(base, CN)], segs)
        pltpu.sync_copy(h_hbm.at[pl.ds(base, CN), :], hbuf)
        pltpu.sync_copy(zp_hbm.at[pl.ds(base, CN), :], zbuf)

        @pl.loop(0, CN, step=2)
        def _(j):
            for b in range(2):
                for g in range(D_H // 16):
                    v = hbuf[j + b, pl.ds(16 * g, 16)]
                    nrows[j + b, pl.ds(16 * g, 16)] = v
                    nrows[j + b, pl.ds(D_H + 16 * g, 16)] = v * v
                zv = zbuf[j + b, :]
                zq = zv * zv
                zqs = _shuffle(zq, sh3)
                nrows[j + b, pl.ds(256, 16)] = zv + zqs + cvec_n

        pltpu.sync_copy(nrows, acc_n.at[segs], add=True)

    plsc.subcore_barrier()
    pltpu.sync_copy(acc_n.at[pl.ds(s * rpt, rpt), :],
                    nacc_out.at[c, pl.ds(s * rpt, rpt), :])


@functools.partial(
    pl.kernel,
    out_type=jax.ShapeDtypeStruct((NC, SEG_PAD, ROW_E), jnp.float32),
    mesh=_mesh,
    compiler_params=pltpu.CompilerParams(use_tc_tiling_on_sc=False),
    scratch_types=[
        pltpu.VMEM((CE,), jnp.int32),
        pltpu.VMEM((CE,), jnp.int32),
        pltpu.VMEM((CE, D_E), jnp.float32),
        pltpu.VMEM((CE, ROW_E), jnp.float32),
        pltpu.VMEM_SHARED((SEG_PAD, ROW_E), jnp.float32),
        pltpu.SemaphoreType.DMA,
    ],
)
def _edge_stats(eid_hbm, bid_hbm, eattr_hbm, ze_hbm, eacc_out,
                eidx, esegs, xbuf, erows, acc_e, sem):
    c = lax.axis_index("c")
    s = lax.axis_index("s")
    rpt = SEG_PAD // NS
    pltpu.sync_copy(ze_hbm.at[pl.ds(s * rpt, rpt), :],
                    acc_e.at[pl.ds(s * rpt, rpt), :])
    iota = lax.iota(jnp.int32, 16)
    cvec_e = jnp.where(iota == 8, 1.0, 0.0).astype(jnp.float32)

    @pl.loop(0, CE)
    def _(j):
        erows[j, pl.ds(24, 16)] = cvec_e

    plsc.subcore_barrier()

    epc = NCHE // NC

    @pl.loop(c * epc + s, (c + 1) * epc, step=NS)
    def _(ch):
        base = ch * CE
        pltpu.sync_copy(eid_hbm.at[pl.ds(base, CE)], eidx)
        pltpu.async_copy(bid_hbm.at[eidx], esegs, sem).wait()
        pltpu.sync_copy(eattr_hbm.at[pl.ds(base, CE), :], xbuf)

        @pl.loop(0, CE, step=8)
        def _(j):
            for b in range(8):
                v = xbuf[j + b, :]
                erows[j + b, pl.ds(0, 16)] = v
                erows[j + b, pl.ds(16, 16)] = v * v

        pltpu.sync_copy(erows, acc_e.at[esegs], add=True)

    plsc.subcore_barrier()
    pltpu.sync_copy(acc_e.at[pl.ds(s * rpt, rpt), :],
                    eacc_out.at[c, pl.ds(s * rpt, rpt), :])


def _finalize_body(nacc_ref, eacc_ref, sig_ref, gh_ref, bh_ref, ge_ref,
                   be_ref, ntab_ref, ztab_ref, etab_ref):
    n2 = nacc_ref[0] + nacc_ref[1]
    S = n2[:, :D_H]
    Q = n2[:, D_H:2 * D_H]
    zS = n2[:, 256:259]
    zQ = n2[:, 259:262]
    n = n2[:, 262:263]
    n1 = jnp.maximum(n, 1.0)
    mu = S / n1
    var = jnp.maximum(Q - n * mu * mu, 0.0) / jnp.maximum(n - 1.0, 1.0)
    sd = jnp.sqrt(var + 1e-12)
    a = gh_ref[...] / (sd + 1e-8)
    cst = bh_ref[...] - mu * a
    ntab_ref[...] = jnp.concatenate([a, cst], axis=1)

    muz = zS / n1
    sqz = jnp.sum(zQ - n * muz * muz, axis=1, keepdims=True)
    varz = jnp.maximum(sqz, 0.0) / jnp.maximum(3.0 * n - 1.0, 1.0)
    var_ez = jnp.sqrt(varz + 1e-12) + 1e-8
    resc = sig_ref[...] / var_ez
    cz = muz * (1.0 - resc)
    ztab_ref[...] = jnp.concatenate(
        [resc, cz, jnp.zeros((SEG_PAD, 10), jnp.float32)], axis=1)

    e2 = eacc_ref[0] + eacc_ref[1]
    se = e2[:, :D_E]
    qe = e2[:, D_E:2 * D_E]
    m = e2[:, 2 * D_E:2 * D_E + 1]
    m1 = jnp.maximum(m, 1.0)
    mue = se / m1
    vare = jnp.maximum(qe - m * mue * mue, 0.0) / jnp.maximum(m - 1.0, 1.0)
    sde = jnp.sqrt(vare + 1e-12)
    ae = ge_ref[...] / (sde + 1e-8)
    ce = be_ref[...] - mue * ae
    etab_ref[...] = jnp.concatenate([ae, ce], axis=1)


_finalize = pl.pallas_call(
    _finalize_body,
    out_shape=[
        jax.ShapeDtypeStruct((SEG_PAD, 2 * D_H), jnp.float32),
        jax.ShapeDtypeStruct((SEG_PAD, 16), jnp.float32),
        jax.ShapeDtypeStruct((SEG_PAD, 2 * D_E), jnp.float32),
    ],
)


@functools.partial(
    pl.kernel,
    out_type=[
        jax.ShapeDtypeStruct((N_NODES, D_H), jnp.float32),
        jax.ShapeDtypeStruct((N_NODES, 16), jnp.float32),
        jax.ShapeDtypeStruct((N_EDGES, D_E), jnp.float32),
    ],
    mesh=_mesh,
    compiler_params=pltpu.CompilerParams(use_tc_tiling_on_sc=False),
    scratch_types=[
        pltpu.VMEM((CN,), jnp.int32),
        pltpu.VMEM((CN, D_H), jnp.float32),
        pltpu.VMEM((CN, 16), jnp.float32),
        pltpu.VMEM((CN, 2 * D_H), jnp.float32),
        pltpu.VMEM((CN, 16), jnp.float32),
        pltpu.VMEM((CN, D_H), jnp.float32),
        pltpu.VMEM((CN, 16), jnp.float32),
        pltpu.VMEM((CE,), jnp.int32),
        pltpu.VMEM((CE,), jnp.int32),
        pltpu.VMEM((CE, D_E), jnp.float32),
        pltpu.VMEM((CE, 2 * D_E), jnp.float32),
        pltpu.VMEM((CE, D_E), jnp.float32),
        pltpu.SemaphoreType.DMA,
        pltpu.SemaphoreType.DMA,
    ],
)
def _norm(h_hbm, zp_hbm, bid_hbm, eid_hbm, eattr_hbm,
          ntab_hbm, ztab_hbm, etab_hbm,
          hout_hbm, zout_hbm, eout_hbm,
          segs, hbuf, zbuf, trows, ztrows, hout, zout,
          eidx, esegs, xbuf, etrows, outb, sem1, sem2):
    c = lax.axis_index("c")
    s = lax.axis_index("s")
    iota = lax.iota(jnp.int32, 16)
    sh3p = (iota + 3) & 15
    npc = NCHN // NC

    @pl.loop(c * npc + s, (c + 1) * npc, step=NS)
    def _(ch):
        base = ch * CN
        pltpu.sync_copy(bid_hbm.at[pl.ds(base, CN)], segs)
        cp1 = pltpu.async_copy(ntab_hbm.at[segs], trows, sem1)
        cp2 = pltpu.async_copy(ztab_hbm.at[segs], ztrows, sem2)
        pltpu.sync_copy(h_hbm.at[pl.ds(base, CN), :], hbuf)
        pltpu.sync_copy(zp_hbm.at[pl.ds(base, CN), :], zbuf)
        cp1.wait()
        cp2.wait()

        @pl.loop(0, CN, step=2)
        def _(j):
            for b in range(2):
                for g in range(D_H // 16):
                    h = hbuf[j + b, pl.ds(16 * g, 16)]
                    aa = trows[j + b, pl.ds(16 * g, 16)]
                    cc = trows[j + b, pl.ds(D_H + 16 * g, 16)]
                    hout[j + b, pl.ds(16 * g, 16)] = h * aa + cc
                zv = zbuf[j + b, :]
                tz = ztrows[j + b, :]
                czs = _shuffle(tz, sh3p)
                zout[j + b, :] = zv * tz + czs

        pltpu.sync_copy(hout, hout_hbm.at[pl.ds(base, CN), :])
        pltpu.sync_copy(zout, zout_hbm.at[pl.ds(base, CN), :])

    epc = NCHE // NC

    @pl.loop(c * epc + s, (c + 1) * epc, step=NS)
    def _(ch):
        base = ch * CE
        pltpu.sync_copy(eid_hbm.at[pl.ds(base, CE)], eidx)
        pltpu.async_copy(bid_hbm.at[eidx], esegs, sem1).wait()
        cp = pltpu.async_copy(etab_hbm.at[esegs], etrows, sem1)
        pltpu.sync_copy(eattr_hbm.at[pl.ds(base, CE), :], xbuf)
        cp.wait()

        @pl.loop(0, CE, step=8)
        def _(j):
            for b in range(8):
                x = xbuf[j + b, :]
                aa = etrows[j + b, pl.ds(0, D_E)]
                cc = etrows[j + b, pl.ds(D_E, D_E)]
                outb[j + b, :] = x * aa + cc

        pltpu.sync_copy(outb, eout_hbm.at[pl.ds(base, CE), :])


def kernel(H, Z, edge_attr, block_id, edge_id, sigma, gamma_H, beta_H,
           gamma_E, beta_E):
    Zp = jnp.pad(Z, ((0, 0), (0, 13)))
    eid0 = edge_id[0]
    zn = jnp.zeros((SEG_PAD, ROW_N), jnp.float32)
    ze = jnp.zeros((SEG_PAD, ROW_E), jnp.float32)

    nacc = _node_stats(H, Zp, block_id, zn)
    eacc = _edge_stats(eid0, block_id, edge_attr, ze)

    ntab, ztab, etab = _finalize(
        nacc, eacc,
        sigma.reshape(1, 3),
        gamma_H.reshape(1, -1), beta_H.reshape(1, -1),
        gamma_E.reshape(1, -1), beta_E.reshape(1, -1),
    )

    H_out, Zp_out, edge_out = _norm(
        H, Zp, block_id, eid0, edge_attr, ntab, ztab, etab)
    rescale = ztab[:N_SEG, :3]
    return (H_out, Zp_out[:, :3], edge_out, rescale)


# speedup vs baseline: 1.3325x; 1.3325x over previous
import functools

import jax
import jax.numpy as jnp
from jax import lax
from jax.experimental import pallas as pl
from jax.experimental.pallas import tpu as pltpu
from jax.experimental.pallas import tpu_sc as plsc

N_NODES = 100000
N_EDGES = 1600000
D_H = 128
D_E = 16
N_SEG = 5000
SEG_PAD = 5120
NC, NS = 2, 16
CN = 80
NCHN = N_NODES // CN
CE = 640
NCHE = N_EDGES // CE

ROW_N = 272
ROW_E = 40

_mesh = plsc.VectorSubcoreMesh(core_axis_name="c", subcore_axis_name="s")

_GATHER_DNUMS = lax.GatherDimensionNumbers(
    offset_dims=(), collapsed_slice_dims=(0,), start_index_map=(0,))


def _shuffle(x, idx):
    return lax.gather(x, idx[:, None], _GATHER_DNUMS, (1,),
                      mode=lax.GatherScatterMode.PROMISE_IN_BOUNDS)


@functools.partial(
    pl.kernel,
    out_type=jax.ShapeDtypeStruct((NC, SEG_PAD, ROW_N), jnp.float32),
    mesh=_mesh,
    compiler_params=pltpu.CompilerParams(use_tc_tiling_on_sc=False),
    scratch_types=[
        pltpu.VMEM((CN,), jnp.int32),
        pltpu.VMEM((CN, D_H), jnp.float32),
        pltpu.VMEM((CN, 16), jnp.float32),
        pltpu.VMEM((CN, ROW_N), jnp.float32),
        pltpu.VMEM_SHARED((SEG_PAD, ROW_N), jnp.float32),
        pltpu.SemaphoreType.DMA,
    ],
)
def _node_stats(h_hbm, zp_hbm, bid_hbm, zn_hbm, nacc_out,
                segs, hbuf, zbuf, nrows, acc_n, sem):
    c = lax.axis_index("c")
    s = lax.axis_index("s")
    rpt = SEG_PAD // NS
    pltpu.sync_copy(zn_hbm.at[pl.ds(s * rpt, rpt), :],
                    acc_n.at[pl.ds(s * rpt, rpt), :])
    iota = lax.iota(jnp.int32, 16)
    cvec_n = jnp.where(iota == 6, 1.0, 0.0).astype(jnp.float32)
    sh3 = (iota - 3) & 15

    plsc.subcore_barrier()

    npc = NCHN // NC

    @pl.loop(c * npc + s, (c + 1) * npc, step=NS)
    def _(ch):
        base = ch * CN
        pltpu.sync_copy(bid_hbm.at[pl.ds(base, CN)], segs)
        pltpu.sync_copy(h_hbm.at[pl.ds(base, CN), :], hbuf)
        pltpu.sync_copy(zp_hbm.at[pl.ds(base, CN), :], zbuf)

        @pl.loop(0, CN, step=2)
        def _(j):
            for b in range(2):
                for g in range(D_H // 16):
                    v = hbuf[j + b, pl.ds(16 * g, 16)]
                    nrows[j + b, pl.ds(16 * g, 16)] = v
                    nrows[j + b, pl.ds(D_H + 16 * g, 16)] = v * v
                zv = zbuf[j + b, :]
                zq = zv * zv
                zqs = _shuffle(zq, sh3)
                nrows[j + b, pl.ds(256, 16)] = zv + zqs + cvec_n

        pltpu.sync_copy(nrows, acc_n.at[segs], add=True)

    plsc.subcore_barrier()
    pltpu.sync_copy(acc_n.at[pl.ds(s * rpt, rpt), :],
                    nacc_out.at[c, pl.ds(s * rpt, rpt), :])


@functools.partial(
    pl.kernel,
    out_type=jax.ShapeDtypeStruct((NC, SEG_PAD, ROW_E), jnp.float32),
    mesh=_mesh,
    compiler_params=pltpu.CompilerParams(use_tc_tiling_on_sc=False),
    scratch_types=[
        pltpu.VMEM((CE,), jnp.int32),
        pltpu.VMEM((CE,), jnp.int32),
        pltpu.VMEM((CE, D_E), jnp.float32),
        pltpu.VMEM((CE, ROW_E), jnp.float32),
        pltpu.VMEM_SHARED((SEG_PAD, ROW_E), jnp.float32),
        pltpu.SemaphoreType.DMA,
    ],
)
def _edge_stats(eid_hbm, bid_hbm, eattr_hbm, ze_hbm, eacc_out,
                eidx, esegs, xbuf, erows, acc_e, sem):
    c = lax.axis_index("c")
    s = lax.axis_index("s")
    rpt = SEG_PAD // NS
    pltpu.sync_copy(ze_hbm.at[pl.ds(s * rpt, rpt), :],
                    acc_e.at[pl.ds(s * rpt, rpt), :])
    iota = lax.iota(jnp.int32, 16)
    cvec_e = jnp.where(iota == 8, 1.0, 0.0).astype(jnp.float32)

    @pl.loop(0, CE)
    def _(j):
        erows[j, pl.ds(24, 16)] = cvec_e

    plsc.subcore_barrier()

    epc = NCHE // NC

    @pl.loop(c * epc + s, (c + 1) * epc, step=NS)
    def _(ch):
        base = ch * CE
        pltpu.sync_copy(eid_hbm.at[pl.ds(base, CE)], eidx)
        pltpu.async_copy(bid_hbm.at[eidx], esegs, sem).wait()
        pltpu.sync_copy(eattr_hbm.at[pl.ds(base, CE), :], xbuf)

        @pl.loop(0, CE, step=8)
        def _(j):
            for b in range(8):
                v = xbuf[j + b, :]
                erows[j + b, pl.ds(0, 16)] = v
                erows[j + b, pl.ds(16, 16)] = v * v

        pltpu.sync_copy(erows, acc_e.at[esegs], add=True)

    plsc.subcore_barrier()
    pltpu.sync_copy(acc_e.at[pl.ds(s * rpt, rpt), :],
                    eacc_out.at[c, pl.ds(s * rpt, rpt), :])


def _finalize_body(nacc_ref, eacc_ref, sig_ref, gh_ref, bh_ref, ge_ref,
                   be_ref, ntab_ref, ztab_ref, etab_ref):
    n2 = nacc_ref[0] + nacc_ref[1]
    S = n2[:, :D_H]
    Q = n2[:, D_H:2 * D_H]
    zS = n2[:, 256:259]
    zQ = n2[:, 259:262]
    n = n2[:, 262:263]
    n1 = jnp.maximum(n, 1.0)
    mu = S / n1
    var = jnp.maximum(Q - n * mu * mu, 0.0) / jnp.maximum(n - 1.0, 1.0)
    sd = jnp.sqrt(var + 1e-12)
    a = gh_ref[...] / (sd + 1e-8)
    cst = bh_ref[...] - mu * a
    ntab_ref[...] = jnp.concatenate([a, cst], axis=1)

    muz = zS / n1
    sqz = jnp.sum(zQ - n * muz * muz, axis=1, keepdims=True)
    varz = jnp.maximum(sqz, 0.0) / jnp.maximum(3.0 * n - 1.0, 1.0)
    var_ez = jnp.sqrt(varz + 1e-12) + 1e-8
    resc = sig_ref[...] / var_ez
    cz = muz * (1.0 - resc)
    ztab_ref[...] = jnp.concatenate(
        [resc, cz, jnp.zeros((SEG_PAD, 10), jnp.float32)], axis=1)

    e2 = eacc_ref[0] + eacc_ref[1]
    se = e2[:, :D_E]
    qe = e2[:, D_E:2 * D_E]
    m = e2[:, 2 * D_E:2 * D_E + 1]
    m1 = jnp.maximum(m, 1.0)
    mue = se / m1
    vare = jnp.maximum(qe - m * mue * mue, 0.0) / jnp.maximum(m - 1.0, 1.0)
    sde = jnp.sqrt(vare + 1e-12)
    ae = ge_ref[...] / (sde + 1e-8)
    ce = be_ref[...] - mue * ae
    etab_ref[...] = jnp.concatenate([ae, ce], axis=1)


_finalize = pl.pallas_call(
    _finalize_body,
    out_shape=[
        jax.ShapeDtypeStruct((SEG_PAD, 2 * D_H), jnp.float32),
        jax.ShapeDtypeStruct((SEG_PAD, 16), jnp.float32),
        jax.ShapeDtypeStruct((SEG_PAD, 2 * D_E), jnp.float32),
    ],
)


@functools.partial(
    pl.kernel,
    out_type=[
        jax.ShapeDtypeStruct((N_NODES, D_H), jnp.float32),
        jax.ShapeDtypeStruct((N_NODES, 16), jnp.float32),
        jax.ShapeDtypeStruct((N_EDGES, D_E), jnp.float32),
    ],
    mesh=_mesh,
    compiler_params=pltpu.CompilerParams(use_tc_tiling_on_sc=False),
    scratch_types=[
        pltpu.VMEM((CN,), jnp.int32),
        pltpu.VMEM((CN, D_H), jnp.float32),
        pltpu.VMEM((CN, 16), jnp.float32),
        pltpu.VMEM((CN, 2 * D_H), jnp.float32),
        pltpu.VMEM((CN, 16), jnp.float32),
        pltpu.VMEM((CN, D_H), jnp.float32),
        pltpu.VMEM((CN, 16), jnp.float32),
        pltpu.VMEM((CE,), jnp.int32),
        pltpu.VMEM((CE,), jnp.int32),
        pltpu.VMEM((CE, D_E), jnp.float32),
        pltpu.VMEM((CE, 2 * D_E), jnp.float32),
        pltpu.VMEM((CE, D_E), jnp.float32),
        pltpu.SemaphoreType.DMA,
        pltpu.SemaphoreType.DMA,
    ],
)
def _norm(h_hbm, zp_hbm, bid_hbm, eid_hbm, eattr_hbm,
          ntab_hbm, ztab_hbm, etab_hbm,
          hout_hbm, zout_hbm, eout_hbm,
          segs, hbuf, zbuf, trows, ztrows, hout, zout,
          eidx, esegs, xbuf, etrows, outb, sem1, sem2):
    c = lax.axis_index("c")
    s = lax.axis_index("s")
    iota = lax.iota(jnp.int32, 16)
    sh3p = (iota + 3) & 15
    npc = NCHN // NC

    @pl.loop(c * npc + s, (c + 1) * npc, step=NS)
    def _(ch):
        base = ch * CN
        pltpu.sync_copy(bid_hbm.at[pl.ds(base, CN)], segs)
        cp1 = pltpu.async_copy(ntab_hbm.at[segs], trows, sem1)
        cp2 = pltpu.async_copy(ztab_hbm.at[segs], ztrows, sem2)
        pltpu.sync_copy(h_hbm.at[pl.ds(base, CN), :], hbuf)
        pltpu.sync_copy(zp_hbm.at[pl.ds(base, CN), :], zbuf)
        cp1.wait()
        cp2.wait()

        @pl.loop(0, CN, step=2)
        def _(j):
            for b in range(2):
                for g in range(D_H // 16):
                    h = hbuf[j + b, pl.ds(16 * g, 16)]
                    aa = trows[j + b, pl.ds(16 * g, 16)]
                    cc = trows[j + b, pl.ds(D_H + 16 * g, 16)]
                    hout[j + b, pl.ds(16 * g, 16)] = h * aa + cc
                zv = zbuf[j + b, :]
                tz = ztrows[j + b, :]
                czs = _shuffle(tz, sh3p)
                zout[j + b, :] = zv * tz + czs

        pltpu.sync_copy(hout, hout_hbm.at[pl.ds(base, CN), :])
        pltpu.sync_copy(zout, zout_hbm.at[pl.ds(base, CN), :])

    epc = NCHE // NC

    @pl.loop(c * epc + s, (c + 1) * epc, step=NS)
    def _(ch):
        base = ch * CE
        pltpu.sync_copy(eid_hbm.at[pl.ds(base, CE)], eidx)
        pltpu.async_copy(bid_hbm.at[eidx], esegs, sem1).wait()
        cp = pltpu.async_copy(etab_hbm.at[esegs], etrows, sem1)
        pltpu.sync_copy(eattr_hbm.at[pl.ds(base, CE), :], xbuf)
        cp.wait()

        @pl.loop(0, CE, step=8)
        def _(j):
            for b in range(8):
                x = xbuf[j + b, :]
                aa = etrows[j + b, pl.ds(0, D_E)]
                cc = etrows[j + b, pl.ds(D_E, D_E)]
                outb[j + b, :] = x * aa + cc

        pltpu.sync_copy(outb, eout_hbm.at[pl.ds(base, CE), :])


def kernel(H, Z, edge_attr, block_id, edge_id, sigma, gamma_H, beta_H,
           gamma_E, beta_E):
    Zp = jnp.pad(Z, ((0, 0), (0, 13)))
    eid0 = edge_id[0]
    zn = jnp.zeros((SEG_PAD, ROW_N), jnp.float32)
    ze = jnp.zeros((SEG_PAD, ROW_E), jnp.float32)

    nacc = _node_stats(H, Zp, block_id, zn)
    eacc = _edge_stats(eid0, block_id, edge_attr, ze)

    ntab, ztab, etab = _finalize(
        nacc, eacc,
        sigma.reshape(1, 3),
        gamma_H.reshape(1, -1), beta_H.reshape(1, -1),
        gamma_E.reshape(1, -1), beta_E.reshape(1, -1),
    )

    H_out, Zp_out, edge_out = _norm(
        H, Zp, block_id, eid0, edge_attr, ntab, ztab, etab)
    rescale = ztab[:N_SEG, :3]
    return (H_out, Zp_out[:, :3], edge_out, rescale)


# speedup vs baseline: 1.3819x; 1.0370x over previous
import functools

import jax
import jax.numpy as jnp
from jax import lax
from jax.experimental import pallas as pl
from jax.experimental.pallas import tpu as pltpu
from jax.experimental.pallas import tpu_sc as plsc

N_NODES = 100000
N_EDGES = 1600000
D_H = 128
D_E = 16
N_SEG = 5000
SEG_PAD = 5120
NC, NS = 2, 16
CN = 80
NCHN = N_NODES // CN
CE = 1000
NCHE = N_EDGES // CE

ROW_N = 272
ROW_E = 40

_mesh = plsc.VectorSubcoreMesh(core_axis_name="c", subcore_axis_name="s")

_GATHER_DNUMS = lax.GatherDimensionNumbers(
    offset_dims=(), collapsed_slice_dims=(0,), start_index_map=(0,))


def _shuffle(x, idx):
    return lax.gather(x, idx[:, None], _GATHER_DNUMS, (1,),
                      mode=lax.GatherScatterMode.PROMISE_IN_BOUNDS)


@functools.partial(
    pl.kernel,
    out_type=jax.ShapeDtypeStruct((NC, SEG_PAD, ROW_N), jnp.float32),
    mesh=_mesh,
    compiler_params=pltpu.CompilerParams(use_tc_tiling_on_sc=False),
    scratch_types=[
        pltpu.VMEM((CN,), jnp.int32),
        pltpu.VMEM((CN, D_H), jnp.float32),
        pltpu.VMEM((CN, 16), jnp.float32),
        pltpu.VMEM((CN, ROW_N), jnp.float32),
        pltpu.VMEM_SHARED((SEG_PAD, ROW_N), jnp.float32),
        pltpu.SemaphoreType.DMA,
    ],
)
def _node_stats(h_hbm, zp_hbm, bid_hbm, zn_hbm, nacc_out,
                segs, hbuf, zbuf, nrows, acc_n, sem):
    c = lax.axis_index("c")
    s = lax.axis_index("s")
    rpt = SEG_PAD // NS
    pltpu.sync_copy(zn_hbm.at[pl.ds(s * rpt, rpt), :],
                    acc_n.at[pl.ds(s * rpt, rpt), :])
    iota = lax.iota(jnp.int32, 16)
    cvec_n = jnp.where(iota == 6, 1.0, 0.0).astype(jnp.float32)
    sh3 = (iota - 3) & 15

    plsc.subcore_barrier()

    npc = NCHN // NC

    @pl.loop(c * npc + s, (c + 1) * npc, step=NS)
    def _(ch):
        base = ch * CN
        pltpu.sync_copy(bid_hbm.at[pl.ds(base, CN)], segs)
        pltpu.sync_copy(h_hbm.at[pl.ds(base, CN), :], hbuf)
        pltpu.sync_copy(zp_hbm.at[pl.ds(base, CN), :], zbuf)

        @pl.loop(0, CN, step=2)
        def _(j):
            for b in range(2):
                for g in range(D_H // 16):
                    v = hbuf[j + b, pl.ds(16 * g, 16)]
                    nrows[j + b, pl.ds(16 * g, 16)] = v
                    nrows[j + b, pl.ds(D_H + 16 * g, 16)] = v * v
                zv = zbuf[j + b, :]
                zq = zv * zv
                zqs = _shuffle(zq, sh3)
                nrows[j + b, pl.ds(256, 16)] = zv + zqs + cvec_n

        pltpu.sync_copy(nrows, acc_n.at[segs], add=True)

    plsc.subcore_barrier()
    pltpu.sync_copy(acc_n.at[pl.ds(s * rpt, rpt), :],
                    nacc_out.at[c, pl.ds(s * rpt, rpt), :])


@functools.partial(
    pl.kernel,
    out_type=jax.ShapeDtypeStruct((NC, SEG_PAD, ROW_E), jnp.float32),
    mesh=_mesh,
    compiler_params=pltpu.CompilerParams(use_tc_tiling_on_sc=False),
    scratch_types=[
        pltpu.VMEM((CE,), jnp.int32),
        pltpu.VMEM((CE,), jnp.int32),
        pltpu.VMEM((CE, D_E), jnp.float32),
        pltpu.VMEM((CE, ROW_E), jnp.float32),
        pltpu.VMEM_SHARED((SEG_PAD, ROW_E), jnp.float32),
        pltpu.SemaphoreType.DMA,
    ],
)
def _edge_stats(eid_hbm, bid_hbm, eattr_hbm, ze_hbm, eacc_out,
                eidx, esegs, xbuf, erows, acc_e, sem):
    c = lax.axis_index("c")
    s = lax.axis_index("s")
    rpt = SEG_PAD // NS
    pltpu.sync_copy(ze_hbm.at[pl.ds(s * rpt, rpt), :],
                    acc_e.at[pl.ds(s * rpt, rpt), :])
    iota = lax.iota(jnp.int32, 16)
    cvec_e = jnp.where(iota == 8, 1.0, 0.0).astype(jnp.float32)

    @pl.loop(0, CE)
    def _(j):
        erows[j, pl.ds(24, 16)] = cvec_e

    plsc.subcore_barrier()

    epc = NCHE // NC

    @pl.loop(c * epc + s, (c + 1) * epc, step=NS)
    def _(ch):
        base = ch * CE
        pltpu.sync_copy(eid_hbm.at[pl.ds(base, CE)], eidx)
        pltpu.async_copy(bid_hbm.at[eidx], esegs, sem).wait()
        pltpu.sync_copy(eattr_hbm.at[pl.ds(base, CE), :], xbuf)

        @pl.loop(0, CE, step=8)
        def _(j):
            for b in range(8):
                v = xbuf[j + b, :]
                erows[j + b, pl.ds(0, 16)] = v
                erows[j + b, pl.ds(16, 16)] = v * v

        pltpu.sync_copy(erows, acc_e.at[esegs], add=True)

    plsc.subcore_barrier()
    pltpu.sync_copy(acc_e.at[pl.ds(s * rpt, rpt), :],
                    eacc_out.at[c, pl.ds(s * rpt, rpt), :])


def _finalize_body(nacc_ref, eacc_ref, sig_ref, gh_ref, bh_ref, ge_ref,
                   be_ref, ntab_ref, ztab_ref, etab_ref):
    n2 = nacc_ref[0] + nacc_ref[1]
    S = n2[:, :D_H]
    Q = n2[:, D_H:2 * D_H]
    zS = n2[:, 256:259]
    zQ = n2[:, 259:262]
    n = n2[:, 262:263]
    n1 = jnp.maximum(n, 1.0)
    mu = S / n1
    var = jnp.maximum(Q - n * mu * mu, 0.0) / jnp.maximum(n - 1.0, 1.0)
    sd = jnp.sqrt(var + 1e-12)
    a = gh_ref[...] / (sd + 1e-8)
    cst = bh_ref[...] - mu * a
    ntab_ref[...] = jnp.concatenate([a, cst], axis=1)

    muz = zS / n1
    sqz = jnp.sum(zQ - n * muz * muz, axis=1, keepdims=True)
    varz = jnp.maximum(sqz, 0.0) / jnp.maximum(3.0 * n - 1.0, 1.0)
    var_ez = jnp.sqrt(varz + 1e-12) + 1e-8
    resc = sig_ref[...] / var_ez
    cz = muz * (1.0 - resc)
    ztab_ref[...] = jnp.concatenate(
        [resc, cz, jnp.zeros((SEG_PAD, 10), jnp.float32)], axis=1)

    e2 = eacc_ref[0] + eacc_ref[1]
    se = e2[:, :D_E]
    qe = e2[:, D_E:2 * D_E]
    m = e2[:, 2 * D_E:2 * D_E + 1]
    m1 = jnp.maximum(m, 1.0)
    mue = se / m1
    vare = jnp.maximum(qe - m * mue * mue, 0.0) / jnp.maximum(m - 1.0, 1.0)
    sde = jnp.sqrt(vare + 1e-12)
    ae = ge_ref[...] / (sde + 1e-8)
    ce = be_ref[...] - mue * ae
    etab_ref[...] = jnp.concatenate([ae, ce], axis=1)


_finalize = pl.pallas_call(
    _finalize_body,
    out_shape=[
        jax.ShapeDtypeStruct((SEG_PAD, 2 * D_H), jnp.float32),
        jax.ShapeDtypeStruct((SEG_PAD, 16), jnp.float32),
        jax.ShapeDtypeStruct((SEG_PAD, 2 * D_E), jnp.float32),
    ],
)


@functools.partial(
    pl.kernel,
    out_type=[
        jax.ShapeDtypeStruct((N_NODES, D_H), jnp.float32),
        jax.ShapeDtypeStruct((N_NODES, 16), jnp.float32),
        jax.ShapeDtypeStruct((N_EDGES, D_E), jnp.float32),
    ],
    mesh=_mesh,
    compiler_params=pltpu.CompilerParams(use_tc_tiling_on_sc=False),
    scratch_types=[
        pltpu.VMEM((CN,), jnp.int32),
        pltpu.VMEM((CN, D_H), jnp.float32),
        pltpu.VMEM((CN, 16), jnp.float32),
        pltpu.VMEM((CN, 2 * D_H), jnp.float32),
        pltpu.VMEM((CN, 16), jnp.float32),
        pltpu.VMEM((CN, D_H), jnp.float32),
        pltpu.VMEM((CN, 16), jnp.float32),
        pltpu.VMEM((CE,), jnp.int32),
        pltpu.VMEM((CE,), jnp.int32),
        pltpu.VMEM((CE, D_E), jnp.float32),
        pltpu.VMEM((CE, 2 * D_E), jnp.float32),
        pltpu.VMEM((CE, D_E), jnp.float32),
        pltpu.SemaphoreType.DMA,
        pltpu.SemaphoreType.DMA,
    ],
)
def _norm(h_hbm, zp_hbm, bid_hbm, eid_hbm, eattr_hbm,
          ntab_hbm, ztab_hbm, etab_hbm,
          hout_hbm, zout_hbm, eout_hbm,
          segs, hbuf, zbuf, trows, ztrows, hout, zout,
          eidx, esegs, xbuf, etrows, outb, sem1, sem2):
    c = lax.axis_index("c")
    s = lax.axis_index("s")
    iota = lax.iota(jnp.int32, 16)
    sh3p = (iota + 3) & 15
    npc = NCHN // NC

    @pl.loop(c * npc + s, (c + 1) * npc, step=NS)
    def _(ch):
        base = ch * CN
        pltpu.sync_copy(bid_hbm.at[pl.ds(base, CN)], segs)
        cp1 = pltpu.async_copy(ntab_hbm.at[segs], trows, sem1)
        cp2 = pltpu.async_copy(ztab_hbm.at[segs], ztrows, sem2)
        pltpu.sync_copy(h_hbm.at[pl.ds(base, CN), :], hbuf)
        pltpu.sync_copy(zp_hbm.at[pl.ds(base, CN), :], zbuf)
        cp1.wait()
        cp2.wait()

        @pl.loop(0, CN, step=2)
        def _(j):
            for b in range(2):
                for g in range(D_H // 16):
                    h = hbuf[j + b, pl.ds(16 * g, 16)]
                    aa = trows[j + b, pl.ds(16 * g, 16)]
                    cc = trows[j + b, pl.ds(D_H + 16 * g, 16)]
                    hout[j + b, pl.ds(16 * g, 16)] = h * aa + cc
                zv = zbuf[j + b, :]
                tz = ztrows[j + b, :]
                czs = _shuffle(tz, sh3p)
                zout[j + b, :] = zv * tz + czs

        pltpu.sync_copy(hout, hout_hbm.at[pl.ds(base, CN), :])
        pltpu.sync_copy(zout, zout_hbm.at[pl.ds(base, CN), :])

    epc = NCHE // NC

    @pl.loop(c * epc + s, (c + 1) * epc, step=NS)
    def _(ch):
        base = ch * CE
        pltpu.sync_copy(eid_hbm.at[pl.ds(base, CE)], eidx)
        pltpu.async_copy(bid_hbm.at[eidx], esegs, sem1).wait()
        cp = pltpu.async_copy(etab_hbm.at[esegs], etrows, sem1)
        pltpu.sync_copy(eattr_hbm.at[pl.ds(base, CE), :], xbuf)
        cp.wait()

        @pl.loop(0, CE, step=8)
        def _(j):
            for b in range(8):
                x = xbuf[j + b, :]
                aa = etrows[j + b, pl.ds(0, D_E)]
                cc = etrows[j + b, pl.ds(D_E, D_E)]
                outb[j + b, :] = x * aa + cc

        pltpu.sync_copy(outb, eout_hbm.at[pl.ds(base, CE), :])


def kernel(H, Z, edge_attr, block_id, edge_id, sigma, gamma_H, beta_H,
           gamma_E, beta_E):
    Zp = jnp.pad(Z, ((0, 0), (0, 13)))
    eid0 = edge_id[0]
    zn = jnp.zeros((SEG_PAD, ROW_N), jnp.float32)
    ze = jnp.zeros((SEG_PAD, ROW_E), jnp.float32)

    nacc = _node_stats(H, Zp, block_id, zn)
    eacc = _edge_stats(eid0, block_id, edge_attr, ze)

    ntab, ztab, etab = _finalize(
        nacc, eacc,
        sigma.reshape(1, 3),
        gamma_H.reshape(1, -1), beta_H.reshape(1, -1),
        gamma_E.reshape(1, -1), beta_E.reshape(1, -1),
    )

    H_out, Zp_out, edge_out = _norm(
        H, Zp, block_id, eid0, edge_attr, ntab, ztab, etab)
    rescale = ztab[:N_SEG, :3]
    return (H_out, Zp_out[:, :3], edge_out, rescale)
